# Initial kernel scaffold; baseline (speedup 1.0000x reference)
import jax
import jax.numpy as jnp
from jax.experimental import pallas as pl


def kernel(x, edge_index, W, b):
    raise NotImplementedError("write your pallas kernel here")


# speedup vs baseline: 87.8039x; 87.8039x over previous
import functools

import jax
import jax.numpy as jnp
from jax import lax
from jax.experimental import pallas as pl
from jax.experimental.pallas import tpu as pltpu
from jax.experimental.pallas import tpu_sc as plsc

NN = 10000
DD = 128
EE = 320000

NC = 2
NS = 16
NW = NC * NS
LL = 16

NPAD = 10240
SEGF = NPAD // NS

ETOT = EE + NN
EPAD = ((ETOT + NW * LL - 1) // (NW * LL)) * (NW * LL)
EPT = EPAD // NW
NV = EPT // LL

_MESH = dict(core_axis_name="c", subcore_axis_name="s",
             num_cores=NC, num_subcores=NS)
_SC_PARAMS = pltpu.CompilerParams(needs_layout_passes=False)


def _reduce_emit(acc, shared, out_hbm, cid, sid, seg_acc, seg_tmp):
    pltpu.sync_copy(acc, shared.at[sid])
    plsc.subcore_barrier()
    off = sid * SEGF
    pltpu.sync_copy(shared.at[0, pl.ds(off, SEGF)], seg_acc)

    def tile_body(t, _):
        pltpu.sync_copy(shared.at[t, pl.ds(off, SEGF)], seg_tmp)

        def add_body(i, _):
            seg_acc[pl.ds(i * LL, LL)] = (
                seg_acc[pl.ds(i * LL, LL)] + seg_tmp[pl.ds(i * LL, LL)])
            return 0

        lax.fori_loop(0, SEGF // LL, add_body, 0)
        return 0

    lax.fori_loop(1, NS, tile_body, 0)
    pltpu.sync_copy(seg_acc, out_hbm.at[cid, pl.ds(off, SEGF)])


def _zero_vmem(ref, nwords):
    z = jnp.zeros((LL,), jnp.float32)

    def body(i, _):
        ref[pl.ds(i * LL, LL)] = z
        return 0

    lax.fori_loop(0, nwords // LL, body, 0)


@functools.partial(
    pl.kernel,
    out_type=jax.ShapeDtypeStruct((NC, NPAD), jnp.float32),
    mesh=plsc.VectorSubcoreMesh(**_MESH),
    compiler_params=_SC_PARAMS,
    scratch_types=[
        pltpu.VMEM((EPT,), jnp.int32),
        pltpu.VMEM((NPAD,), jnp.float32),
        pltpu.VMEM((SEGF,), jnp.float32),
        pltpu.VMEM((SEGF,), jnp.float32),
        pltpu.VMEM_SHARED((NS, NPAD), jnp.float32),
    ],
)
def _deg_kernel(col_hbm, out_hbm, colv, acc, seg_acc, seg_tmp, shared):
    cid = lax.axis_index("c")
    sid = lax.axis_index("s")
    wid = cid * NS + sid
    pltpu.sync_copy(col_hbm.at[pl.ds(wid * EPT, EPT)], colv)
    _zero_vmem(acc, NPAD)
    ones = jnp.ones((LL,), jnp.float32)

    def edge_body(i, _):
        c = colv[pl.ds(i * LL, LL)]
        plsc.addupdate_scatter(acc, [c], ones)
        return 0

    lax.fori_loop(0, NV, edge_body, 0)
    _reduce_emit(acc, shared, out_hbm, cid, sid, seg_acc, seg_tmp)


def _make_hop(num_partials):
    @functools.partial(
        pl.kernel,
        out_type=jax.ShapeDtypeStruct((NC, NPAD), jnp.float32),
        mesh=plsc.VectorSubcoreMesh(**_MESH),
        compiler_params=_SC_PARAMS,
        scratch_types=[
            pltpu.VMEM((EPT,), jnp.int32),
            pltpu.VMEM((EPT,), jnp.int32),
            pltpu.VMEM((NPAD,), jnp.float32),
            pltpu.VMEM((NPAD,), jnp.float32),
            pltpu.VMEM((NPAD,), jnp.float32),
            pltpu.VMEM((NPAD,), jnp.float32),
            pltpu.VMEM((SEGF,), jnp.float32),
            pltpu.VMEM((SEGF,), jnp.float32),
            pltpu.VMEM_SHARED((NS, NPAD), jnp.float32),
        ],
    )
    def hop_kernel(yp_hbm, row_hbm, col_hbm, dinv_hbm, out_hbm,
                   rowv, colv, ytbl, dtbl, acc, ptmp, seg_acc, seg_tmp,
                   shared):
        cid = lax.axis_index("c")
        sid = lax.axis_index("s")
        wid = cid * NS + sid
        base = wid * EPT
        pltpu.sync_copy(row_hbm.at[pl.ds(base, EPT)], rowv)
        pltpu.sync_copy(col_hbm.at[pl.ds(base, EPT)], colv)
        pltpu.sync_copy(dinv_hbm.at[0], dtbl)
        pltpu.sync_copy(yp_hbm.at[0], ytbl)
        for p in range(1, num_partials):
            pltpu.sync_copy(yp_hbm.at[p], ptmp)

            def psum_body(i, _):
                ytbl[pl.ds(i * LL, LL)] = (
                    ytbl[pl.ds(i * LL, LL)] + ptmp[pl.ds(i * LL, LL)])
                return 0

            lax.fori_loop(0, NPAD // LL, psum_body, 0)
        _zero_vmem(acc, NPAD)

        def edge_body(i, _):
            r = rowv[pl.ds(i * LL, LL)]
            c = colv[pl.ds(i * LL, LL)]
            dr = plsc.load_gather(dtbl, [r])
            dc = plsc.load_gather(dtbl, [c])
            yv = plsc.load_gather(ytbl, [r])
            plsc.addupdate_scatter(acc, [c], dr * dc * yv)
            return 0

        lax.fori_loop(0, NV, edge_body, 0)
        _reduce_emit(acc, shared, out_hbm, cid, sid, seg_acc, seg_tmp)

    return hop_kernel


_hop1 = _make_hop(1)
_hop2 = _make_hop(NC)


def _tc_head(degp, xpad, w):
    def body(degp_ref, x_ref, w_ref, dinv_ref, y_ref):
        deg = degp_ref[0:1, :] + degp_ref[1:2, :]
        dinv_ref[:] = lax.rsqrt(jnp.maximum(deg, 1.0))
        y_ref[:] = jnp.dot(x_ref[:], w_ref[:],
                           preferred_element_type=jnp.float32)

    return pl.pallas_call(
        body,
        out_shape=(jax.ShapeDtypeStruct((1, NPAD), jnp.float32),
                   jax.ShapeDtypeStruct((NPAD, 1), jnp.float32)),
    )(degp, xpad, w)


def _tc_tail(z2, b2d):
    def body(z_ref, b_ref, o_ref):
        o_ref[:] = z_ref[0:1, :NN] + z_ref[1:2, :NN] + b_ref[:]

    return pl.pallas_call(
        body,
        out_shape=jax.ShapeDtypeStruct((1, NN), jnp.float32),
    )(z2, b2d)


def kernel(x, edge_index, W, b):
    loop_idx = jnp.arange(NN, dtype=jnp.int32)
    row = jnp.concatenate([edge_index[0].astype(jnp.int32), loop_idx])
    col = jnp.concatenate([edge_index[1].astype(jnp.int32), loop_idx])
    row = jnp.pad(row, (0, EPAD - ETOT), constant_values=NN)
    col = jnp.pad(col, (0, EPAD - ETOT), constant_values=NN)
    xpad = jnp.pad(x, ((0, NPAD - NN), (0, 0)))

    degp = _deg_kernel(col)
    dinv, y = _tc_head(degp, xpad, W)
    z1 = _hop1(y.reshape(1, NPAD), row, col, dinv)
    z2 = _hop2(z1, row, col, dinv)
    out = _tc_tail(z2, b.reshape(1, 1))
    return out.reshape(NN)
